# baseline (speedup 1.0000x reference)
import functools

import jax
import jax.numpy as jnp
from jax.experimental import pallas as pl
from jax.experimental.pallas import tpu as pltpu

_N = 4096
_L = 200
_C = 128
_G = 8
_NT = _L // _G


def _sort_key(t):
    s = jax.lax.bitcast_convert_type(t, jnp.int32)
    m = jax.lax.shift_right_arithmetic(s, 31)
    return jax.lax.bitcast_convert_type(s ^ (m | jnp.int32(-2147483648)),
                                        jnp.uint32)


def _body(pt_ref, tt_ref, out_ref, kap_ref, e_ref, tmat_ref):
    i = pl.program_id(0)
    p = pt_ref[...]
    t = tt_ref[...]
    m = jnp.max(p, axis=0, keepdims=True)
    e_ref[: _L, :] = jnp.exp(p - m)
    sum_p = jnp.sum(p, axis=0, keepdims=True)
    kap_ref[: _L, :] = _sort_key(t)

    iota8 = jax.lax.broadcasted_iota(jnp.uint32, (_G, _C), 0)

    def group(g, carry):
        base = g * _G
        kq = kap_ref[pl.ds(base, _G), :]
        eq8 = e_ref[pl.ds(base, _G), :]

        kqb = [jnp.broadcast_to(kq[j:j + 1, :], (_G, _C)) for j in range(_G)]

        accs = [jnp.zeros((_G, _C), jnp.float32) for _ in range(_G)]
        for tile in range(_NT):
            lhs = kap_ref[tile * _G:(tile + 1) * _G, :]
            et = e_ref[tile * _G:(tile + 1) * _G, :]
            for j in range(_G):
                accs[j] = accs[j] + jnp.where(lhs > kqb[j], et, 0.0)

        tks = []
        for j in range(_G):
            corr = jnp.where((kq == kqb[j]) & (iota8 <= j), eq8, 0.0)
            tks.append(jnp.sum(accs[j] + corr, axis=0, keepdims=True))
        tmat_ref[pl.ds(base, _G), :] = jnp.concatenate(tks, axis=0)

        kap_ref[pl.ds(base, _G), :] = kq + jnp.uint32(1)
        return carry

    jax.lax.fori_loop(0, _NT, group, 0)

    acc = jnp.sum(jnp.log(tmat_ref[: _L, :]), axis=0, keepdims=True)
    col_loss = -sum_p + acc + jnp.float32(_L) * m
    partial = jnp.sum(col_loss) * jnp.float32(1.0 / _N)

    @pl.when(i == 0)
    def _():
        out_ref[...] = jnp.zeros_like(out_ref)

    out_ref[...] += jnp.full((1, 1), partial, jnp.float32)


@jax.jit
def kernel(y_pred, y_true):
    pt = y_pred.T
    tt = y_true.T
    out = pl.pallas_call(
        _body,
        grid=(_N // _C,),
        in_specs=[
            pl.BlockSpec((_L, _C), lambda i: (0, i)),
            pl.BlockSpec((_L, _C), lambda i: (0, i)),
        ],
        out_specs=pl.BlockSpec((1, 1), lambda i: (0, 0)),
        out_shape=jax.ShapeDtypeStruct((1, 1), jnp.float32),
        scratch_shapes=[
            pltpu.VMEM((_L, _C), jnp.uint32),
            pltpu.VMEM((_L, _C), jnp.float32),
            pltpu.VMEM((_L, _C), jnp.float32),
        ],
    )(pt, tt)
    return out[0, 0]


# speedup vs baseline: 1.0740x; 1.0740x over previous
import functools

import jax
import jax.numpy as jnp
from jax.experimental import pallas as pl
from jax.experimental.pallas import tpu as pltpu

_N = 4096
_L = 200
_C = 256
_G = 8
_NT = _L // _G


def _sort_key(t):
    s = jax.lax.bitcast_convert_type(t, jnp.int32)
    m = jax.lax.shift_right_arithmetic(s, 31)
    return jax.lax.bitcast_convert_type(s ^ (m | jnp.int32(-2147483648)),
                                        jnp.uint32)


def _body(pt_ref, tt_ref, out_ref, kap_ref, e_ref, tmat_ref):
    i = pl.program_id(0)
    p = pt_ref[...]
    t = tt_ref[...]
    m = jnp.max(p, axis=0, keepdims=True)
    e_ref[: _L, :] = jnp.exp(p - m)
    sum_p = jnp.sum(p, axis=0, keepdims=True)
    kap_ref[: _L, :] = _sort_key(t)

    iota8 = jax.lax.broadcasted_iota(jnp.uint32, (_G, _C), 0)

    def group(g, carry):
        base = g * _G
        kq = kap_ref[pl.ds(base, _G), :]
        eq8 = e_ref[pl.ds(base, _G), :]

        tks = [None] * _G
        for half in range(2):
            js = list(range(half * 4, half * 4 + 4))
            kqb = {j: jnp.broadcast_to(kq[j:j + 1, :], (_G, _C)) for j in js}
            accs = {j: jnp.zeros((_G, _C), jnp.float32) for j in js}
            for tile in range(_NT):
                lhs = kap_ref[tile * _G:(tile + 1) * _G, :]
                et = e_ref[tile * _G:(tile + 1) * _G, :]
                for j in js:
                    accs[j] = accs[j] + jnp.where(lhs > kqb[j], et, 0.0)
            for j in js:
                corr = jnp.where((kq == kqb[j]) & (iota8 <= j), eq8, 0.0)
                tks[j] = jnp.sum(accs[j] + corr, axis=0, keepdims=True)
        tmat_ref[pl.ds(base, _G), :] = jnp.concatenate(tks, axis=0)

        kap_ref[pl.ds(base, _G), :] = kq + jnp.uint32(1)
        return carry

    jax.lax.fori_loop(0, _NT, group, 0)

    acc = jnp.sum(jnp.log(tmat_ref[: _L, :]), axis=0, keepdims=True)
    col_loss = -sum_p + acc + jnp.float32(_L) * m
    partial = jnp.sum(col_loss) * jnp.float32(1.0 / _N)

    @pl.when(i == 0)
    def _():
        out_ref[...] = jnp.zeros_like(out_ref)

    out_ref[...] += jnp.full((1, 1), partial, jnp.float32)


@jax.jit
def kernel(y_pred, y_true):
    pt = y_pred.T
    tt = y_true.T
    out = pl.pallas_call(
        _body,
        grid=(_N // _C,),
        in_specs=[
            pl.BlockSpec((_L, _C), lambda i: (0, i)),
            pl.BlockSpec((_L, _C), lambda i: (0, i)),
        ],
        out_specs=pl.BlockSpec((1, 1), lambda i: (0, 0)),
        out_shape=jax.ShapeDtypeStruct((1, 1), jnp.float32),
        scratch_shapes=[
            pltpu.VMEM((_L, _C), jnp.uint32),
            pltpu.VMEM((_L, _C), jnp.float32),
            pltpu.VMEM((_L, _C), jnp.float32),
        ],
    )(pt, tt)
    return out[0, 0]


# speedup vs baseline: 1.0861x; 1.0112x over previous
import functools

import jax
import jax.numpy as jnp
from jax.experimental import pallas as pl
from jax.experimental.pallas import tpu as pltpu

_N = 4096
_L = 200
_C = 256
_G = 8
_NT = _L // _G


def _sort_key(t):
    s = jax.lax.bitcast_convert_type(t, jnp.int32)
    m = jax.lax.shift_right_arithmetic(s, 31)
    return jax.lax.bitcast_convert_type(s ^ (m | jnp.int32(-2147483648)),
                                        jnp.uint32)


def _body(pt_ref, tt_ref, out_ref, kap_ref, e_ref, tmat_ref):
    i = pl.program_id(0)
    p = pt_ref[...]
    t = tt_ref[...]
    m = jnp.max(p, axis=0, keepdims=True)
    e_ref[: _L, :] = jnp.exp(p - m)
    sum_p = jnp.sum(p, axis=0, keepdims=True)
    kap_ref[: _L, :] = _sort_key(t)

    iota8 = jax.lax.broadcasted_iota(jnp.uint32, (_G, _C), 0)

    def group(g, carry):
        base = g * _G
        kq = kap_ref[pl.ds(base, _G), :]
        eq8 = e_ref[pl.ds(base, _G), :]

        kqb = [jnp.broadcast_to(kq[j:j + 1, :], (_G, _C)) for j in range(_G)]

        accs = [jnp.zeros((_G, _C), jnp.float32) for _ in range(_G)]
        for tile in range(_NT):
            lhs = kap_ref[tile * _G:(tile + 1) * _G, :]
            et = e_ref[tile * _G:(tile + 1) * _G, :]
            for j in range(_G):
                accs[j] = accs[j] + jnp.where(lhs > kqb[j], et, 0.0)

        tks = []
        for j in range(_G):
            corr = jnp.where((kq == kqb[j]) & (iota8 <= j), eq8, 0.0)
            tks.append(jnp.sum(accs[j] + corr, axis=0, keepdims=True))
        tmat_ref[pl.ds(base, _G), :] = jnp.concatenate(tks, axis=0)

        kap_ref[pl.ds(base, _G), :] = kq + jnp.uint32(1)
        return carry

    jax.lax.fori_loop(0, _NT, group, 0)

    acc = jnp.sum(jnp.log(tmat_ref[: _L, :]), axis=0, keepdims=True)
    col_loss = -sum_p + acc + jnp.float32(_L) * m
    partial = jnp.sum(col_loss) * jnp.float32(1.0 / _N)

    @pl.when(i == 0)
    def _():
        out_ref[...] = jnp.zeros_like(out_ref)

    out_ref[...] += jnp.full((1, 1), partial, jnp.float32)


@jax.jit
def kernel(y_pred, y_true):
    pt = y_pred.T
    tt = y_true.T
    out = pl.pallas_call(
        _body,
        grid=(_N // _C,),
        in_specs=[
            pl.BlockSpec((_L, _C), lambda i: (0, i)),
            pl.BlockSpec((_L, _C), lambda i: (0, i)),
        ],
        out_specs=pl.BlockSpec((1, 1), lambda i: (0, 0)),
        out_shape=jax.ShapeDtypeStruct((1, 1), jnp.float32),
        scratch_shapes=[
            pltpu.VMEM((_L, _C), jnp.uint32),
            pltpu.VMEM((_L, _C), jnp.float32),
            pltpu.VMEM((_L, _C), jnp.float32),
        ],
    )(pt, tt)
    return out[0, 0]
